# baseline (speedup 1.0000x reference)
import jax
import jax.numpy as jnp
from jax import lax
from jax.experimental import pallas as pl
from jax.experimental.pallas import tpu as pltpu
from jax.experimental.pallas import tpu_sc as plsc

_N_E = 8192
_E_DIM = 256
_N_TOK = 8192
_BETA = 0.25
_TM = 256
_N_STEPS = _N_TOK // _TM


def _vq_body(z_ref, w_ref, kio_ref, d_ref, oh_ref, idx_ref, idxl_ref, loss_ref,
             perp_ref, wt_ref, w2_ref, idxl_acc, counts_ref, dsum_ref):
    i = pl.program_id(0)

    @pl.when(i == 0)
    def _w2():
        wt0 = jnp.transpose(w_ref[...])
        wt_ref[...] = wt0
        w2_ref[...] = jnp.sum(wt0 * wt0, axis=0, keepdims=True)

    z = z_ref[...].reshape(_E_DIM, _TM).T
    mm2 = lax.dot_general(z + z, wt_ref[...], (((1,), (0,)), ((), ())),
                          preferred_element_type=jnp.float32)
    z2 = jnp.sum(z * z, axis=1, keepdims=True)
    d = (z2 + w2_ref[...]) - mm2
    d_ref[...] = d
    m = jnp.min(d, axis=1, keepdims=True)
    kio = kio_ref[...]
    idx_f = jnp.min(jnp.where(d == m, kio, jnp.float32(2.0**30)),
                    axis=1, keepdims=True)
    oh = jnp.where(kio == idx_f, 1.0, 0.0)
    oh_ref[...] = oh
    ii = idx_f.astype(jnp.int32)
    idx_ref[...] = ii
    idxl_acc[pl.ds(2 * i, 1), :] = jnp.transpose(ii[0:128, :])
    idxl_acc[pl.ds(2 * i + 1, 1), :] = jnp.transpose(ii[128:256, :])

    @pl.when(i == 0)
    def _init():
        counts_ref[...] = jnp.zeros_like(counts_ref)
        dsum_ref[...] = jnp.zeros_like(dsum_ref)

    counts_ref[...] += jnp.sum(oh, axis=0, keepdims=True)
    dsum_ref[...] += jnp.sum(m, keepdims=True)

    @pl.when(i == _N_STEPS - 1)
    def _fin():
        loss_ref[...] = dsum_ref[...] * ((1.0 + _BETA) / (_N_TOK * _E_DIM))
        e = counts_ref[...] * (1.0 / _N_TOK)
        ent = jnp.sum(e * jnp.log(e + 1e-10), keepdims=True)
        perp_ref[...] = jnp.exp(-ent)
        idxl_ref[...] = idxl_acc[...]


def _vq_call(z_cmaj, w, kio_row):
    return pl.pallas_call(
        _vq_body,
        grid=(_N_STEPS,),
        in_specs=[
            pl.BlockSpec((1, _E_DIM, 8, 32),
                         lambda i: (i // 4, 0, i % 4, 0)),
            pl.BlockSpec(memory_space=pltpu.VMEM),
            pl.BlockSpec(memory_space=pltpu.VMEM),
        ],
        out_specs=[
            pl.BlockSpec((_TM, _N_E), lambda i: (i, 0)),
            pl.BlockSpec((_TM, _N_E), lambda i: (i, 0)),
            pl.BlockSpec((_TM, 1), lambda i: (i, 0)),
            pl.BlockSpec((_N_TOK // 128, 128), lambda i: (0, 0)),
            pl.BlockSpec((1, 1), lambda i: (0, 0)),
            pl.BlockSpec((1, 1), lambda i: (0, 0)),
        ],
        out_shape=[
            jax.ShapeDtypeStruct((_N_TOK, _N_E), jnp.float32),
            jax.ShapeDtypeStruct((_N_TOK, _N_E), jnp.float32),
            jax.ShapeDtypeStruct((_N_TOK, 1), jnp.int32),
            jax.ShapeDtypeStruct((_N_TOK // 128, 128), jnp.int32),
            jax.ShapeDtypeStruct((1, 1), jnp.float32),
            jax.ShapeDtypeStruct((1, 1), jnp.float32),
        ],
        scratch_shapes=[
            pltpu.VMEM((_E_DIM, _N_E), jnp.float32),
            pltpu.VMEM((1, _N_E), jnp.float32),
            pltpu.VMEM((_N_TOK // 128, 128), jnp.int32),
            pltpu.VMEM((1, _N_E), jnp.float32),
            pltpu.VMEM((1, 1), jnp.float32),
        ],
    )(z_cmaj, w, kio_row)


_NW = 32
_BPW = _N_TOK // _NW


def _gather_body(table_hbm, idx_hbm, out_hbm, idx_v, rows_v, sem):
    wid = lax.axis_index("s") * 2 + lax.axis_index("c")
    base = wid * _BPW
    pltpu.sync_copy(idx_hbm.at[pl.ds(base, _BPW)], idx_v)
    pltpu.async_copy(table_hbm.at[idx_v], rows_v, sem).wait()
    pltpu.sync_copy(rows_v, out_hbm.at[pl.ds(base, _BPW)])


def _sc_gather(W, idx):
    mesh = plsc.VectorSubcoreMesh(core_axis_name="c", subcore_axis_name="s")
    return pl.kernel(
        _gather_body,
        out_type=jax.ShapeDtypeStruct((_N_TOK, _E_DIM), jnp.float32),
        mesh=mesh,
        scratch_types=[
            pltpu.VMEM((_BPW,), jnp.int32),
            pltpu.VMEM((_BPW, _E_DIM), jnp.float32),
            pltpu.SemaphoreType.DMA,
        ],
    )(W, idx)


def kernel(z, W):
    z_cmaj = z
    kio_row = lax.broadcasted_iota(jnp.float32, (1, _N_E), 1)
    d, oh, idx2, idxl, loss11, perp11 = _vq_call(z_cmaj, W, kio_row)
    zq = _sc_gather(W, idxl.reshape(-1))
    z_q_out = jnp.transpose(zq.reshape(8, 32, 32, _E_DIM), (0, 3, 1, 2))
    loss = loss11.reshape(())
    perp = perp11.reshape(())
    return (z_q_out, loss, (perp, oh, idx2, d), W)


# speedup vs baseline: 1.1138x; 1.1138x over previous
import jax
import jax.numpy as jnp
from jax import lax
from jax.experimental import pallas as pl
from jax.experimental.pallas import tpu as pltpu
from jax.experimental.pallas import tpu_sc as plsc

_N_E = 8192
_E_DIM = 256
_N_TOK = 8192
_BETA = 0.25
_TM = 256
_N_STEPS = _N_TOK // _TM


def _vq_body(z_ref, w_ref, kio_ref, d_ref, oh_ref, idx_ref, idxl_ref, loss_ref,
             perp_ref, wt_ref, w2_ref, idxl_acc, counts_ref, dsum_ref):
    i = pl.program_id(0)

    @pl.when(i == 0)
    def _w2():
        wt0 = jnp.transpose(w_ref[...])
        wt_ref[...] = wt0
        w2_ref[...] = jnp.sum(wt0 * wt0, axis=0, keepdims=True)

    z = z_ref[...].reshape(_E_DIM, _TM).T
    mm2 = lax.dot_general(z + z, wt_ref[...], (((1,), (0,)), ((), ())),
                          preferred_element_type=jnp.float32)
    z2 = jnp.sum(z * z, axis=1, keepdims=True)
    d = (z2 + w2_ref[...]) - mm2
    d_ref[...] = d
    m = jnp.min(d, axis=1, keepdims=True)
    kio = kio_ref[...]
    idx_f = jnp.min(jnp.where(d == m, kio, jnp.float32(2.0**30)),
                    axis=1, keepdims=True)
    oh = jnp.where(kio == idx_f, 1.0, 0.0)
    oh_ref[...] = oh
    ii = idx_f.astype(jnp.int32)
    idx_ref[...] = ii
    idxl_acc[pl.ds(2 * i, 1), :] = jnp.transpose(ii[0:128, :])
    idxl_acc[pl.ds(2 * i + 1, 1), :] = jnp.transpose(ii[128:256, :])

    @pl.when(i == 0)
    def _init():
        counts_ref[...] = jnp.zeros_like(counts_ref)
        dsum_ref[...] = jnp.zeros_like(dsum_ref)

    counts_ref[...] += jnp.sum(oh, axis=0, keepdims=True)
    dsum_ref[...] += jnp.sum(m, keepdims=True)

    @pl.when(i == _N_STEPS - 1)
    def _fin():
        loss_ref[...] = dsum_ref[...] * ((1.0 + _BETA) / (_N_TOK * _E_DIM))
        e = counts_ref[...] * (1.0 / _N_TOK)
        ent = jnp.sum(e * jnp.log(e + 1e-10), keepdims=True)
        perp_ref[...] = jnp.exp(-ent)
        idxl_ref[...] = idxl_acc[...]


def _vq_call(z_cmaj, w, kio_row):
    return pl.pallas_call(
        _vq_body,
        grid=(_N_STEPS,),
        in_specs=[
            pl.BlockSpec((1, _E_DIM, _TM),
                         lambda i: (i // (1024 // _TM), 0, i % (1024 // _TM))),
            pl.BlockSpec(memory_space=pltpu.VMEM),
            pl.BlockSpec(memory_space=pltpu.VMEM),
        ],
        out_specs=[
            pl.BlockSpec((_TM, _N_E), lambda i: (i, 0)),
            pl.BlockSpec((_TM, _N_E), lambda i: (i, 0)),
            pl.BlockSpec((_TM, 1), lambda i: (i, 0)),
            pl.BlockSpec((_N_TOK // 128, 128), lambda i: (0, 0)),
            pl.BlockSpec((1, 1), lambda i: (0, 0)),
            pl.BlockSpec((1, 1), lambda i: (0, 0)),
        ],
        out_shape=[
            jax.ShapeDtypeStruct((_N_TOK, _N_E), jnp.float32),
            jax.ShapeDtypeStruct((_N_TOK, _N_E), jnp.float32),
            jax.ShapeDtypeStruct((_N_TOK, 1), jnp.int32),
            jax.ShapeDtypeStruct((_N_TOK // 128, 128), jnp.int32),
            jax.ShapeDtypeStruct((1, 1), jnp.float32),
            jax.ShapeDtypeStruct((1, 1), jnp.float32),
        ],
        scratch_shapes=[
            pltpu.VMEM((_E_DIM, _N_E), jnp.float32),
            pltpu.VMEM((1, _N_E), jnp.float32),
            pltpu.VMEM((_N_TOK // 128, 128), jnp.int32),
            pltpu.VMEM((1, _N_E), jnp.float32),
            pltpu.VMEM((1, 1), jnp.float32),
        ],
    )(z_cmaj, w, kio_row)


_NW = 32
_BPW = _N_TOK // _NW


def _gather_body(table_hbm, idx_hbm, out_hbm, idx_v, rows_v, sem):
    wid = lax.axis_index("s") * 2 + lax.axis_index("c")
    base = wid * _BPW
    pltpu.sync_copy(idx_hbm.at[pl.ds(base, _BPW)], idx_v)
    pltpu.async_copy(table_hbm.at[idx_v], rows_v, sem).wait()
    pltpu.sync_copy(rows_v, out_hbm.at[pl.ds(base, _BPW)])


def _sc_gather(W, idx):
    mesh = plsc.VectorSubcoreMesh(core_axis_name="c", subcore_axis_name="s")
    return pl.kernel(
        _gather_body,
        out_type=jax.ShapeDtypeStruct((_N_TOK, _E_DIM), jnp.float32),
        mesh=mesh,
        scratch_types=[
            pltpu.VMEM((_BPW,), jnp.int32),
            pltpu.VMEM((_BPW, _E_DIM), jnp.float32),
            pltpu.SemaphoreType.DMA,
        ],
    )(W, idx)


def kernel(z, W):
    z_cmaj = z.reshape(8, _E_DIM, 1024)
    kio_row = lax.broadcasted_iota(jnp.float32, (1, _N_E), 1)
    d, oh, idx2, idxl, loss11, perp11 = _vq_call(z_cmaj, W, kio_row)
    zq = _sc_gather(W, idxl.reshape(-1))
    z_q_out = jnp.transpose(zq.reshape(8, 32, 32, _E_DIM), (0, 3, 1, 2))
    loss = loss11.reshape(())
    perp = perp11.reshape(())
    return (z_q_out, loss, (perp, oh, idx2, d), W)
